# baseline (speedup 1.0000x reference)
import functools

import jax
import jax.numpy as jnp
from jax import lax
from jax.experimental import pallas as pl
from jax.experimental.pallas import tpu as pltpu
from jax.experimental.pallas import tpu_sc as plsc

_SC_CORES = 2
_SC_SUBCORES = 16
_NW = _SC_CORES * _SC_SUBCORES
_CHUNK = 128


def _topk_one_level(u, k_ref, i_ref, w1_ref, w2_ref, w3_ref, Q, m):
    ux, uy, uz = u[:, 0:1], u[:, 1:2], u[:, 2:3]
    kx, ky, kz = k_ref[0:1, :], k_ref[1:2, :], k_ref[2:3, :]
    dx = ux - kx
    dy = uy - ky
    dz = uz - kz
    D = dx * dx + dy * dy + dz * dz

    lane0 = jax.lax.broadcasted_iota(jnp.int32, (Q, 128), 1)
    INF = jnp.float32(jnp.inf)
    BIGI = jnp.int32(2 ** 30)

    def run_argmin(excludes):
        best = jnp.full((Q, 128), INF, jnp.float32)
        besti = jnp.zeros((Q, 128), jnp.int32)
        for c in range(m // 128):
            Dc = D[:, c * 128:(c + 1) * 128]
            b = Dc < best
            if excludes:
                flat = lane0 + jnp.int32(c * 128)
                for e in excludes:
                    b = b & (flat != e)
            best = jnp.where(b, Dc, best)
            besti = jnp.where(b, jnp.int32(c), besti)
        mv = jnp.min(best, axis=1, keepdims=True)
        flat = besti * 128 + lane0
        iv = jnp.min(jnp.where(best == mv, flat, BIGI), axis=1, keepdims=True)
        return mv, iv

    m1, i1 = run_argmin(())
    m2, i2 = run_argmin((i1,))
    m3, i3 = run_argmin((i1, i2))

    d1 = jnp.sqrt(m1 + 1e-12)
    d2 = jnp.sqrt(m2 + 1e-12)
    d3 = jnp.sqrt(m3 + 1e-12)
    r1 = 1.0 / (d1 + 1e-8)
    r2 = 1.0 / (d2 + 1e-8)
    r3 = 1.0 / (d3 + 1e-8)
    norm = r1 + r2 + r3

    i_ref[...] = jnp.concatenate([i1, i2, i3], axis=1)
    w1_ref[...] = r1 / norm
    w2_ref[...] = r2 / norm
    w3_ref[...] = r3 / norm


def _topk_all_body(u_ref, k0_ref, k1_ref, k2_ref,
                   i0_ref, wa0, wa1, wa2,
                   i1_ref, wb0, wb1, wb2,
                   i2_ref, wc0, wc1, wc2, *, Q, ms):
    u = u_ref[...]
    _topk_one_level(u, k0_ref, i0_ref, wa0, wa1, wa2, Q, ms[0])
    _topk_one_level(u, k1_ref, i1_ref, wb0, wb1, wb2, Q, ms[1])
    _topk_one_level(u, k2_ref, i2_ref, wc0, wc1, wc2, Q, ms[2])


def _nn_topk_all(unknown, ktp0, ktp1, ktp2, Q):
    N = unknown.shape[0]
    ms = (ktp0.shape[1], ktp1.shape[1], ktp2.shape[1])
    body = functools.partial(_topk_all_body, Q=Q, ms=ms)
    out_level = [
        jax.ShapeDtypeStruct((N, 3), jnp.int32),
        jax.ShapeDtypeStruct((N, 1), jnp.float32),
        jax.ShapeDtypeStruct((N, 1), jnp.float32),
        jax.ShapeDtypeStruct((N, 1), jnp.float32),
    ]
    spec_level = [
        pl.BlockSpec((Q, 3), lambda i: (i, 0)),
        pl.BlockSpec((Q, 1), lambda i: (i, 0)),
        pl.BlockSpec((Q, 1), lambda i: (i, 0)),
        pl.BlockSpec((Q, 1), lambda i: (i, 0)),
    ]
    return pl.pallas_call(
        body,
        grid=(N // Q,),
        in_specs=[
            pl.BlockSpec((Q, 3), lambda i: (i, 0)),
            pl.BlockSpec((8, ms[0]), lambda i: (0, 0)),
            pl.BlockSpec((8, ms[1]), lambda i: (0, 0)),
            pl.BlockSpec((8, ms[2]), lambda i: (0, 0)),
        ],
        out_specs=spec_level * 3,
        out_shape=out_level * 3,
    )(unknown, ktp0, ktp1, ktp2)


def _sc_gather(feat0, feat1, feat2, idx0, idx1, idx2, B):
    bpw = B // _NW
    nch = bpw // _CHUNK
    mesh = plsc.VectorSubcoreMesh(core_axis_name="c", subcore_axis_name="s")
    out_t = [jax.ShapeDtypeStruct((B, 32), jnp.float32)] * 3

    @functools.partial(
        pl.kernel,
        out_type=out_t,
        mesh=mesh,
        compiler_params=pltpu.CompilerParams(use_tc_tiling_on_sc=False),
        scratch_types=[
            pltpu.VMEM((nch, _CHUNK), jnp.int32),
            pltpu.VMEM((bpw, 32), jnp.float32),
            pltpu.SemaphoreType.DMA,
        ],
    )
    def k(f0, f1, f2, i0, i1, i2, o0, o1, o2, idx_v, rows_v, sem):
        wid = lax.axis_index("s") * _SC_CORES + lax.axis_index("c")
        base = wid * bpw
        for f, i, o in ((f0, i0, o0), (f1, i1, o1), (f2, i2, o2)):
            pltpu.sync_copy(i.at[wid], idx_v)
            copies = []
            for j in range(nch):
                copies.append(pltpu.async_copy(
                    f.at[idx_v.at[j]],
                    rows_v.at[pl.ds(j * _CHUNK, _CHUNK)],
                    sem))
            for c in copies:
                c.wait()
            pltpu.sync_copy(rows_v, o.at[pl.ds(base, bpw)])

    return k(feat0, feat1, feat2, idx0, idx1, idx2)


def _head_body(r0_ref, r1_ref, r2_ref,
               wa0, wa1, wa2, wb0, wb1, wb2, wc0, wc1, wc2,
               f0_ref, f1_ref, f2_ref, wc_ref, wr_ref, cls_ref, reg_ref):
    hi = jax.lax.Precision.HIGHEST
    pw = None
    for r_ref, w_refs, f_ref in ((r0_ref, (wa0, wa1, wa2), f0_ref),
                                 (r1_ref, (wb0, wb1, wb2), f1_ref),
                                 (r2_ref, (wc0, wc1, wc2), f2_ref)):
        r = r_ref[...]
        p = (w_refs[0][...] * r[:, 0:32]
             + w_refs[1][...] * r[:, 32:64]
             + w_refs[2][...] * r[:, 64:96])
        t = jnp.dot(p, f_ref[...], preferred_element_type=jnp.float32,
                    precision=hi)
        pw = t if pw is None else pw + t
    cls_ref[...] = jnp.dot(pw, wc_ref[...], preferred_element_type=jnp.float32,
                           precision=hi)
    reg_ref[...] = jnp.dot(pw, wr_ref[...], preferred_element_type=jnp.float32,
                           precision=hi)


def _head(r0, r1, r2, w0, w1, w2, W_fc, W_cls, W_reg):
    N = r0.shape[0]
    QB = 2048
    qb_spec = pl.BlockSpec((QB, 96), lambda i: (i, 0))
    w_spec = pl.BlockSpec((QB, 1), lambda i: (i, 0))
    fc_spec = pl.BlockSpec((32, 64), lambda i: (0, 0))
    cls, reg = pl.pallas_call(
        _head_body,
        grid=(N // QB,),
        in_specs=[qb_spec, qb_spec, qb_spec] + [w_spec] * 9
                 + [fc_spec, fc_spec, fc_spec,
                    pl.BlockSpec((64, 1), lambda i: (0, 0)),
                    pl.BlockSpec((64, 3), lambda i: (0, 0))],
        out_specs=[
            pl.BlockSpec((QB, 1), lambda i: (i, 0)),
            pl.BlockSpec((QB, 3), lambda i: (i, 0)),
        ],
        out_shape=[
            jax.ShapeDtypeStruct((N, 1), jnp.float32),
            jax.ShapeDtypeStruct((N, 3), jnp.float32),
        ],
    )(r0, r1, r2, *w0, *w1, *w2, W_fc[0:32], W_fc[32:64], W_fc[64:96],
      W_cls, W_reg)
    return cls, reg


def _pad_t(known):
    return jnp.pad(known.T, ((0, 5), (0, 0)))


def kernel(unknown, known0, feat0, known1, feat1, known2, feat2, spatial,
           W_fc, W_cls, W_reg):
    N = unknown.shape[0]
    B = N * 3
    outs = _nn_topk_all(unknown, _pad_t(known0), _pad_t(known1),
                        _pad_t(known2), Q=256)
    idx0, w0 = outs[0], outs[1:4]
    idx1, w1 = outs[4], outs[5:8]
    idx2, w2 = outs[8], outs[9:12]
    tile = (_NW, B // _NW // _CHUNK, _CHUNK)
    r0, r1, r2 = _sc_gather(feat0, feat1, feat2,
                            idx0.reshape(tile), idx1.reshape(tile),
                            idx2.reshape(tile), B)
    point_cls, point_reg = _head(r0.reshape(N, 96), r1.reshape(N, 96),
                                 r2.reshape(N, 96), w0, w1, w2,
                                 W_fc, W_cls, W_reg)
    n_, c_, d_, h_, w_ = spatial.shape
    spatial_features = spatial.reshape(n_, c_ * d_, h_, w_)
    return (point_cls, point_reg, spatial_features)


# speedup vs baseline: 1.0050x; 1.0050x over previous
import functools

import jax
import jax.numpy as jnp
from jax import lax
from jax.experimental import pallas as pl
from jax.experimental.pallas import tpu as pltpu
from jax.experimental.pallas import tpu_sc as plsc

_SC_CORES = 2
_SC_SUBCORES = 16
_NW = _SC_CORES * _SC_SUBCORES
_CHUNK = 128


def _topk_one_level(u, k_ref, i_ref, w1_ref, w2_ref, w3_ref, Q, m):
    ux, uy, uz = u[:, 0:1], u[:, 1:2], u[:, 2:3]
    kx, ky, kz = k_ref[0:1, :], k_ref[1:2, :], k_ref[2:3, :]
    dx = ux - kx
    dy = uy - ky
    dz = uz - kz
    D = dx * dx + dy * dy + dz * dz

    lane0 = jax.lax.broadcasted_iota(jnp.int32, (Q, 128), 1)
    INF = jnp.float32(jnp.inf)
    BIGI = jnp.int32(2 ** 30)

    def run_argmin(excludes):
        best = jnp.full((Q, 128), INF, jnp.float32)
        besti = jnp.zeros((Q, 128), jnp.int32)
        for c in range(m // 128):
            Dc = D[:, c * 128:(c + 1) * 128]
            b = Dc < best
            if excludes:
                flat = lane0 + jnp.int32(c * 128)
                for e in excludes:
                    b = b & (flat != e)
            best = jnp.where(b, Dc, best)
            besti = jnp.where(b, jnp.int32(c), besti)
        mv = jnp.min(best, axis=1, keepdims=True)
        flat = besti * 128 + lane0
        iv = jnp.min(jnp.where(best == mv, flat, BIGI), axis=1, keepdims=True)
        return mv, iv

    m1, i1 = run_argmin(())
    m2, i2 = run_argmin((i1,))
    m3, i3 = run_argmin((i1, i2))

    d1 = jnp.sqrt(m1 + 1e-12)
    d2 = jnp.sqrt(m2 + 1e-12)
    d3 = jnp.sqrt(m3 + 1e-12)
    r1 = 1.0 / (d1 + 1e-8)
    r2 = 1.0 / (d2 + 1e-8)
    r3 = 1.0 / (d3 + 1e-8)
    norm = r1 + r2 + r3

    i_ref[...] = jnp.concatenate([i1, i2, i3], axis=1)
    w1_ref[...] = r1 / norm
    w2_ref[...] = r2 / norm
    w3_ref[...] = r3 / norm


def _topk_all_body(u_ref, k0_ref, k1_ref, k2_ref,
                   i0_ref, wa0, wa1, wa2,
                   i1_ref, wb0, wb1, wb2,
                   i2_ref, wc0, wc1, wc2, *, Q, ms):
    u = u_ref[...]
    _topk_one_level(u, k0_ref, i0_ref, wa0, wa1, wa2, Q, ms[0])
    _topk_one_level(u, k1_ref, i1_ref, wb0, wb1, wb2, Q, ms[1])
    _topk_one_level(u, k2_ref, i2_ref, wc0, wc1, wc2, Q, ms[2])


def _nn_topk_all(unknown, ktp0, ktp1, ktp2, Q):
    N = unknown.shape[0]
    ms = (ktp0.shape[1], ktp1.shape[1], ktp2.shape[1])
    body = functools.partial(_topk_all_body, Q=Q, ms=ms)
    out_level = [
        jax.ShapeDtypeStruct((N, 3), jnp.int32),
        jax.ShapeDtypeStruct((N, 1), jnp.float32),
        jax.ShapeDtypeStruct((N, 1), jnp.float32),
        jax.ShapeDtypeStruct((N, 1), jnp.float32),
    ]
    spec_level = [
        pl.BlockSpec((Q, 3), lambda i: (i, 0)),
        pl.BlockSpec((Q, 1), lambda i: (i, 0)),
        pl.BlockSpec((Q, 1), lambda i: (i, 0)),
        pl.BlockSpec((Q, 1), lambda i: (i, 0)),
    ]
    return pl.pallas_call(
        body,
        grid=(N // Q,),
        in_specs=[
            pl.BlockSpec((Q, 3), lambda i: (i, 0)),
            pl.BlockSpec((8, ms[0]), lambda i: (0, 0)),
            pl.BlockSpec((8, ms[1]), lambda i: (0, 0)),
            pl.BlockSpec((8, ms[2]), lambda i: (0, 0)),
        ],
        out_specs=spec_level * 3,
        out_shape=out_level * 3,
    )(unknown, ktp0, ktp1, ktp2)


def _sc_gather(feat0, feat1, feat2, idx0, idx1, idx2, B):
    bpw = B // _NW
    nch = bpw // _CHUNK
    mesh = plsc.VectorSubcoreMesh(core_axis_name="c", subcore_axis_name="s")
    out_t = [jax.ShapeDtypeStruct((B, 32), jnp.float32)] * 3

    @functools.partial(
        pl.kernel,
        out_type=out_t,
        mesh=mesh,
        compiler_params=pltpu.CompilerParams(use_tc_tiling_on_sc=False),
        scratch_types=[
            pltpu.VMEM((nch, _CHUNK), jnp.int32),
            pltpu.VMEM((bpw, 32), jnp.float32),
            pltpu.SemaphoreType.DMA,
        ],
    )
    def k(f0, f1, f2, i0, i1, i2, o0, o1, o2, idx_v, rows_v, sem):
        wid = lax.axis_index("s") * _SC_CORES + lax.axis_index("c")
        base = wid * bpw
        for f, i, o in ((f0, i0, o0), (f1, i1, o1), (f2, i2, o2)):
            pltpu.sync_copy(i.at[wid], idx_v)
            copies = []
            for j in range(nch):
                copies.append(pltpu.async_copy(
                    f.at[idx_v.at[j]],
                    rows_v.at[pl.ds(j * _CHUNK, _CHUNK)],
                    sem))
            for c in copies:
                c.wait()
            pltpu.sync_copy(rows_v, o.at[pl.ds(base, bpw)])

    return k(feat0, feat1, feat2, idx0, idx1, idx2)


def _head_body(r0_ref, r1_ref, r2_ref,
               wa0, wa1, wa2, wb0, wb1, wb2, wc0, wc1, wc2,
               f0_ref, f1_ref, f2_ref, wc_ref, wr_ref, cls_ref, reg_ref):
    hi = jax.lax.Precision.HIGHEST
    pw = None
    for r_ref, w_refs, f_ref in ((r0_ref, (wa0, wa1, wa2), f0_ref),
                                 (r1_ref, (wb0, wb1, wb2), f1_ref),
                                 (r2_ref, (wc0, wc1, wc2), f2_ref)):
        r = r_ref[...]
        p = (w_refs[0][...] * r[:, 0:32]
             + w_refs[1][...] * r[:, 32:64]
             + w_refs[2][...] * r[:, 64:96])
        t = jnp.dot(p, f_ref[...], preferred_element_type=jnp.float32,
                    precision=hi)
        pw = t if pw is None else pw + t
    cls_ref[...] = jnp.dot(pw, wc_ref[...], preferred_element_type=jnp.float32,
                           precision=hi)
    reg_ref[...] = jnp.dot(pw, wr_ref[...], preferred_element_type=jnp.float32,
                           precision=hi)


def _head(r0, r1, r2, w0, w1, w2, W_fc, W_cls, W_reg):
    N = r0.shape[0]
    QB = 2048
    qb_spec = pl.BlockSpec((QB, 96), lambda i: (i, 0))
    w_spec = pl.BlockSpec((QB, 1), lambda i: (i, 0))
    fc_spec = pl.BlockSpec((32, 64), lambda i: (0, 0))
    cls, reg = pl.pallas_call(
        _head_body,
        grid=(N // QB,),
        in_specs=[qb_spec, qb_spec, qb_spec] + [w_spec] * 9
                 + [fc_spec, fc_spec, fc_spec,
                    pl.BlockSpec((64, 1), lambda i: (0, 0)),
                    pl.BlockSpec((64, 3), lambda i: (0, 0))],
        out_specs=[
            pl.BlockSpec((QB, 1), lambda i: (i, 0)),
            pl.BlockSpec((QB, 3), lambda i: (i, 0)),
        ],
        out_shape=[
            jax.ShapeDtypeStruct((N, 1), jnp.float32),
            jax.ShapeDtypeStruct((N, 3), jnp.float32),
        ],
    )(r0, r1, r2, *w0, *w1, *w2, W_fc[0:32], W_fc[32:64], W_fc[64:96],
      W_cls, W_reg)
    return cls, reg


def _pad_t(known):
    return jnp.pad(known.T, ((0, 5), (0, 0)))


def kernel(unknown, known0, feat0, known1, feat1, known2, feat2, spatial,
           W_fc, W_cls, W_reg):
    N = unknown.shape[0]
    B = N * 3
    outs = _nn_topk_all(unknown, _pad_t(known0), _pad_t(known1),
                        _pad_t(known2), Q=128)
    idx0, w0 = outs[0], outs[1:4]
    idx1, w1 = outs[4], outs[5:8]
    idx2, w2 = outs[8], outs[9:12]
    tile = (_NW, B // _NW // _CHUNK, _CHUNK)
    r0, r1, r2 = _sc_gather(feat0, feat1, feat2,
                            idx0.reshape(tile), idx1.reshape(tile),
                            idx2.reshape(tile), B)
    point_cls, point_reg = _head(r0.reshape(N, 96), r1.reshape(N, 96),
                                 r2.reshape(N, 96), w0, w1, w2,
                                 W_fc, W_cls, W_reg)
    n_, c_, d_, h_, w_ = spatial.shape
    spatial_features = spatial.reshape(n_, c_ * d_, h_, w_)
    return (point_cls, point_reg, spatial_features)
